# Initial kernel scaffold; baseline (speedup 1.0000x reference)
import jax
import jax.numpy as jnp
from jax.experimental import pallas as pl


def kernel(x, edge_index, W1, b1, W2, b2, Wc, bc):
    raise NotImplementedError("write your pallas kernel here")


# speedup vs baseline: 26.5136x; 26.5136x over previous
import functools

import jax
import jax.numpy as jnp
from jax import lax
from jax.experimental import pallas as pl
from jax.experimental.pallas import tpu as pltpu
from jax.experimental.pallas import tpu_sc as plsc

_NC, _NS = 2, 16
_NW = _NC * _NS
_CHUNK = 128


def _cdiv(a, b):
    return (a + b - 1) // b


def _mesh():
    return plsc.VectorSubcoreMesh(core_axis_name="c", subcore_axis_name="s")


def _make_deg_kernel(n_pad, ch):
    rows = n_pad // _NS

    @functools.partial(
        pl.kernel,
        out_type=jax.ShapeDtypeStruct((_NC * n_pad,), jnp.float32),
        mesh=_mesh(),
        scratch_types=[
            pltpu.VMEM((ch, _CHUNK), jnp.int32),
            pltpu.VMEM((_CHUNK,), jnp.float32),
            pltpu.VMEM((rows,), jnp.float32),
            pltpu.VMEM_SHARED((n_pad,), jnp.float32),
        ],
        compiler_params=pltpu.CompilerParams(use_tc_tiling_on_sc=False),
    )
    def deg_kernel(dst_hbm, zeros_hbm, out_hbm, idx_v, ones_v, bounce, acc):
        c = lax.axis_index("c")
        s = lax.axis_index("s")
        wid = c * _NS + s
        pltpu.sync_copy(zeros_hbm.at[pl.ds(s * rows, rows)], bounce)
        pltpu.sync_copy(bounce, acc.at[pl.ds(s * rows, rows)])
        for i in range(_CHUNK // 16):
            ones_v[pl.ds(i * 16, 16)] = jnp.ones((16,), jnp.float32)
        pltpu.sync_copy(dst_hbm.at[wid], idx_v)
        plsc.subcore_barrier()

        def body(j, carry):
            pltpu.sync_copy(ones_v, acc.at[idx_v.at[j]], add=True)
            return carry

        lax.fori_loop(0, ch, body, 0)
        plsc.subcore_barrier()
        pltpu.sync_copy(acc.at[pl.ds(s * rows, rows)], bounce)
        pltpu.sync_copy(bounce, out_hbm.at[pl.ds(c * n_pad + s * rows, rows)])

    return deg_kernel


def _make_agg_kernel(n_pad, ch, d):
    rows = n_pad // _NS

    @functools.partial(
        pl.kernel,
        out_type=jax.ShapeDtypeStruct((_NC, n_pad, d), jnp.float32),
        mesh=_mesh(),
        scratch_types=[
            pltpu.VMEM((ch, _CHUNK), jnp.int32),
            pltpu.VMEM((ch, _CHUNK), jnp.int32),
            pltpu.VMEM((_CHUNK, d), jnp.float32),
            pltpu.VMEM((rows, d), jnp.float32),
            pltpu.SemaphoreType.DMA,
            pltpu.VMEM_SHARED((n_pad, d), jnp.float32),
        ],
        compiler_params=pltpu.CompilerParams(use_tc_tiling_on_sc=False),
    )
    def agg_kernel(g_hbm, src_hbm, dst_hbm, zeros_hbm, out_hbm,
                   src_v, dst_v, rows_v, bounce, sem, acc):
        c = lax.axis_index("c")
        s = lax.axis_index("s")
        wid = c * _NS + s
        pltpu.sync_copy(zeros_hbm.at[pl.ds(s * rows, rows)], bounce)
        pltpu.sync_copy(bounce, acc.at[pl.ds(s * rows, rows)])
        pltpu.sync_copy(src_hbm.at[wid], src_v)
        pltpu.sync_copy(dst_hbm.at[wid], dst_v)
        plsc.subcore_barrier()

        def body(j, carry):
            pltpu.async_copy(g_hbm.at[src_v.at[j]], rows_v, sem).wait()
            pltpu.sync_copy(rows_v, acc.at[dst_v.at[j]], add=True)
            return carry

        lax.fori_loop(0, ch, body, 0)
        plsc.subcore_barrier()
        pltpu.sync_copy(acc.at[pl.ds(s * rows, rows)], bounce)
        pltpu.sync_copy(bounce, out_hbm.at[c, pl.ds(s * rows, rows)])

    return agg_kernel


def _scale_mm_body(x_ref, w_ref, degt_ref, o_ref):
    d = degt_ref[...]
    dis = lax.rsqrt(d[:, 0:1] + d[:, 1:2] + 1.0)
    xw = jnp.dot(x_ref[...], w_ref[...], preferred_element_type=jnp.float32)
    o_ref[...] = xw * dis


def _fused_mid_body(aggp_ref, g_ref, degt_ref, b_ref, w_ref, o_ref):
    d = degt_ref[...]
    dis = lax.rsqrt(d[:, 0:1] + d[:, 1:2] + 1.0)
    aggsum = aggp_ref[0] + aggp_ref[1] + g_ref[...]
    h = jnp.maximum(aggsum * dis + b_ref[...], 0.0)
    o_ref[...] = jnp.dot(h, w_ref[...], preferred_element_type=jnp.float32) * dis


def _fused_out_body(aggp_ref, g_ref, degt_ref, b_ref, wc_ref, bc_ref, o_ref):
    d = degt_ref[...]
    dis = lax.rsqrt(d[:, 0:1] + d[:, 1:2] + 1.0)
    h = jnp.maximum((aggp_ref[0] + aggp_ref[1] + g_ref[...]) * dis + b_ref[...],
                    0.0)
    o_ref[...] = (jnp.dot(h, wc_ref[...], preferred_element_type=jnp.float32)
                  + bc_ref[...])


def kernel(x, edge_index, W1, b1, W2, b2, Wc, bc):
    n, d_in = x.shape
    e = edge_index.shape[1]
    d = W1.shape[1]
    d_out = Wc.shape[1]

    ch = _cdiv(e, _NW * _CHUNK)
    e_pad = _NW * ch * _CHUNK
    n_pad = _cdiv(n + 1, _NS * 8) * _NS * 8

    rb = 2000 if n % 2000 == 0 else n
    grid = (n // rb,)

    e32 = edge_index.astype(jnp.int32)
    pad = e_pad - e
    src_p = jnp.concatenate(
        [e32[0], jnp.zeros((pad,), jnp.int32)]).reshape(_NW, ch, _CHUNK)
    dst_p = jnp.concatenate(
        [e32[1], jnp.full((pad,), n, jnp.int32)]).reshape(_NW, ch, _CHUNK)
    zeros_deg = jnp.zeros((n_pad,), jnp.float32)
    zeros_big = jnp.zeros((n_pad, d), jnp.float32)

    deg_parts = _make_deg_kernel(n_pad, ch)(dst_p, zeros_deg).reshape(_NC, n_pad)
    degt = deg_parts[:, :n].T

    g1 = pl.pallas_call(
        _scale_mm_body,
        grid=grid,
        in_specs=[
            pl.BlockSpec((rb, d_in), lambda i: (i, 0)),
            pl.BlockSpec((d_in, d), lambda i: (0, 0)),
            pl.BlockSpec((rb, _NC), lambda i: (i, 0)),
        ],
        out_specs=pl.BlockSpec((rb, d), lambda i: (i, 0)),
        out_shape=jax.ShapeDtypeStruct((n, d), jnp.float32),
    )(x, W1, degt)

    agg = _make_agg_kernel(n_pad, ch, d)
    a1 = agg(g1, src_p, dst_p, zeros_big)[:, :n, :]

    g2 = pl.pallas_call(
        _fused_mid_body,
        grid=grid,
        in_specs=[
            pl.BlockSpec((_NC, rb, d), lambda i: (0, i, 0)),
            pl.BlockSpec((rb, d), lambda i: (i, 0)),
            pl.BlockSpec((rb, _NC), lambda i: (i, 0)),
            pl.BlockSpec((1, d), lambda i: (0, 0)),
            pl.BlockSpec((d, d), lambda i: (0, 0)),
        ],
        out_specs=pl.BlockSpec((rb, d), lambda i: (i, 0)),
        out_shape=jax.ShapeDtypeStruct((n, d), jnp.float32),
    )(a1, g1, degt, b1.reshape(1, d), W2)

    a2 = agg(g2, src_p, dst_p, zeros_big)[:, :n, :]

    y = pl.pallas_call(
        _fused_out_body,
        grid=grid,
        in_specs=[
            pl.BlockSpec((_NC, rb, d), lambda i: (0, i, 0)),
            pl.BlockSpec((rb, d), lambda i: (i, 0)),
            pl.BlockSpec((rb, _NC), lambda i: (i, 0)),
            pl.BlockSpec((1, d), lambda i: (0, 0)),
            pl.BlockSpec((d, d_out), lambda i: (0, 0)),
            pl.BlockSpec((1, d_out), lambda i: (0, 0)),
        ],
        out_specs=pl.BlockSpec((rb, d_out), lambda i: (i, 0)),
        out_shape=jax.ShapeDtypeStruct((n, d_out), jnp.float32),
    )(a2, g2, degt, b2.reshape(1, d), Wc, bc.reshape(1, d_out))

    return y
